# baseline (speedup 1.0000x reference)
import functools

import jax
import jax.numpy as jnp
from jax import lax
from jax.experimental import pallas as pl
from jax.experimental.pallas import tpu as pltpu
from jax.experimental.pallas import tpu_sc as plsc

_E = 16
_SEQ = 20
_NC, _NS = 2, 16
_NW = _NC * _NS
_CHUNK = 128
_WBUF = 2560
_BA = 512
_G = 8
_BB = 512
_NSTATIC = 13


def _sc_gather_body(word_table, city_table, truck_table, lcl_table,
                    handling_table, security_table, range_table, scene_table,
                    cargo_table, word_idx, c0, c1, t0, t1, t2, t3, t4,
                    s_lcl, s_han, s_sec, s_rng, s_scn, s_cgo,
                    word_out, *rest):
  static_outs = rest[:_NSTATIC]
  widx_v, wrows_v, sidx_v, srows_v, sem = rest[_NSTATIC:]
  b = static_outs[0].shape[0]
  r = b // _NW
  rw = (b * _SEQ) // _NW
  wid = lax.axis_index("s") * _NC + lax.axis_index("c")
  base = wid * r
  wbase = wid * rw

  tasks = [(city_table, c0), (city_table, c1),
           (truck_table, t0), (truck_table, t1), (truck_table, t2),
           (truck_table, t3), (truck_table, t4),
           (lcl_table, s_lcl), (handling_table, s_han),
           (security_table, s_sec), (range_table, s_rng),
           (scene_table, s_scn), (cargo_table, s_cgo)]
  for (tab, idx_hbm), st_out in zip(tasks, static_outs):
    pltpu.sync_copy(idx_hbm.at[pl.ds(base, r)], sidx_v)
    cps = []
    for j in range(r // _CHUNK):
      cps.append(pltpu.async_copy(
          tab.at[sidx_v.at[pl.ds(j * _CHUNK, _CHUNK)]],
          srows_v.at[pl.ds(j * _CHUNK, _CHUNK)], sem))
    for c in cps:
      c.wait()
    pltpu.sync_copy(srows_v, st_out.at[pl.ds(base, r)])

  pltpu.sync_copy(word_idx.at[pl.ds(wbase, rw)], widx_v)

  def body(g, carry):
    off = g * _WBUF
    cps = []
    for j in range(_WBUF // _CHUNK):
      cps.append(pltpu.async_copy(
          word_table.at[widx_v.at[pl.ds(off + j * _CHUNK, _CHUNK)]],
          wrows_v.at[pl.ds(j * _CHUNK, _CHUNK)], sem))
    for c in cps:
      c.wait()
    pltpu.sync_copy(wrows_v, word_out.at[pl.ds(wbase + off, _WBUF)])
    return carry

  lax.fori_loop(0, rw // _WBUF, body, 0)


def _sc_gather(b, word_table, city_table, truck_table, lcl_table,
               handling_table, security_table, range_table, scene_table,
               cargo_table, word_idx, c0, c1, t0, t1, t2, t3, t4,
               s_lcl, s_han, s_sec, s_rng, s_scn, s_cgo):
  r = b // _NW
  rw = (b * _SEQ) // _NW
  fn = pl.kernel(
      _sc_gather_body,
      out_type=[jax.ShapeDtypeStruct((b * _SEQ, _E), jnp.float32)] +
               [jax.ShapeDtypeStruct((b, _E), jnp.float32)] * _NSTATIC,
      mesh=plsc.VectorSubcoreMesh(core_axis_name="c", subcore_axis_name="s"),
      scratch_types=[
          pltpu.VMEM((rw,), jnp.int32),
          pltpu.VMEM((_WBUF, _E), jnp.float32),
          pltpu.VMEM((r,), jnp.int32),
          pltpu.VMEM((r, _E), jnp.float32),
          pltpu.SemaphoreType.DMA,
      ],
      compiler_params=pltpu.CompilerParams(use_tc_tiling_on_sc=False),
  )
  return fn(word_table, city_table, truck_table, lcl_table, handling_table,
            security_table, range_table, scene_table, cargo_table,
            word_idx, c0, c1, t0, t1, t2, t3, t4,
            s_lcl, s_han, s_sec, s_rng, s_scn, s_cgo)


def _layer_norm(x, g, bta):
  m = jnp.mean(x, axis=-1, keepdims=True)
  d = x - m
  v = jnp.mean(d * d, axis=-1, keepdims=True)
  return d / jnp.sqrt(v + 1e-5) * g + bta


def _tf_body(h_ref, wqkv_ref, wo_ref, ln1g_ref, ln1b_ref,
             ln2g_ref, ln2b_ref, wf1_ref, bf1_ref, wf2_ref, bf2_ref,
             out_ref, q_s, k_s, v_s, e_s):
  h = h_ref[...]
  qkv = jnp.dot(h, wqkv_ref[...], preferred_element_type=jnp.float32)
  q_s[...] = qkv[:, 0:_E]
  k_s[...] = qkv[:, _E:2 * _E]
  v_s[...] = qkv[:, 2 * _E:3 * _E]

  gl = _G * _SEQ
  n = _BA * _SEQ
  ngrp = _BA // _G

  def sbody(g, carry):
    off = g * gl
    e_s[pl.ds(off, gl), :] = lax.dot_general(
        q_s[pl.ds(off, gl), :], k_s[pl.ds(off, gl), :],
        (((1,), (1,)), ((), ())), preferred_element_type=jnp.float32)
    return carry

  lax.fori_loop(0, ngrp, sbody, 0, unroll=4)

  ri = (lax.broadcasted_iota(jnp.int32, (n, gl), 0) // _SEQ) % _G
  ci = lax.broadcasted_iota(jnp.int32, (n, gl), 1) // _SEQ
  e = jnp.where(ri == ci, jnp.exp(e_s[...]), 0.0)
  r = 1.0 / jnp.sum(e, axis=-1, keepdims=True)
  e_s[...] = e

  def obody(g, carry):
    off = g * gl
    q_s[pl.ds(off, gl), :] = jnp.dot(
        e_s[pl.ds(off, gl), :], v_s[pl.ds(off, gl), :],
        preferred_element_type=jnp.float32)
    return carry

  lax.fori_loop(0, ngrp, obody, 0, unroll=4)

  o = jnp.dot(q_s[...] * r, wo_ref[...], preferred_element_type=jnp.float32)
  h1 = _layer_norm(h + o, ln1g_ref[...], ln1b_ref[...])
  f = jnp.maximum(
      jnp.dot(h1, wf1_ref[...], preferred_element_type=jnp.float32)
      + bf1_ref[...], 0.0)
  f = jnp.dot(f, wf2_ref[...], preferred_element_type=jnp.float32) + bf2_ref[...]
  out_ref[...] = _layer_norm(h1 + f, ln2g_ref[...], ln2b_ref[...])


def _transformer(b, h, wqkv, wo, ln1g, ln1b, ln2g, ln2b,
                 wf1, bf1, wf2, bf2):
  n = _BA * _SEQ
  wspec = lambda shp: pl.BlockSpec(shp, lambda i: (0, 0))
  return pl.pallas_call(
      _tf_body,
      grid=(b // _BA,),
      in_specs=[
          pl.BlockSpec((n, _E), lambda i: (i, 0)),
          wspec((_E, 3 * _E)), wspec((_E, _E)),
          wspec((1, _E)), wspec((1, _E)), wspec((1, _E)), wspec((1, _E)),
          wspec((_E, 64)), wspec((1, 64)), wspec((64, _E)), wspec((1, _E)),
      ],
      out_specs=pl.BlockSpec((n, _E), lambda i: (i, 0)),
      out_shape=jax.ShapeDtypeStruct((b * _SEQ, _E), jnp.float32),
      scratch_shapes=[pltpu.VMEM((n, _E), jnp.float32),
                      pltpu.VMEM((n, _E), jnp.float32),
                      pltpu.VMEM((n, _E), jnp.float32),
                      pltpu.VMEM((n, _G * _SEQ), jnp.float32)],
  )(h, wqkv, wo, ln1g, ln1b, ln2g, ln2b, wf1, bf1, wf2, bf2)


def _tower_body(x_ref, c0_ref, c1_ref, t0_ref, t1_ref, t2_ref, t3_ref, t4_ref,
                lcl_ref, han_ref, sec_ref, rng_ref, scn_ref, cgo_ref,
                de_ref, wl_ref, bl_ref, fmv_ref, fmv2_ref,
                w1_ref, b1_ref, w2_ref, b2_ref, y_ref):
  scale = 0.125
  num = jnp.dot(x_ref[...], wl_ref[...] * scale,
                preferred_element_type=jnp.float32) + bl_ref[...]
  tm = (t0_ref[...] + t1_ref[...] + t2_ref[...] + t3_ref[...]
        + t4_ref[...]) * 0.2
  de = de_ref[...]
  out = jnp.concatenate(
      [num, c0_ref[...], c1_ref[...], tm, lcl_ref[...], han_ref[...],
       sec_ref[...], rng_ref[...], scn_ref[...], cgo_ref[...], de],
      axis=1)
  s = jnp.dot(out, fmv_ref[...], preferred_element_type=jnp.float32)
  t2 = jnp.dot(out * out, fmv2_ref[...], preferred_element_type=jnp.float32)
  fm = 0.5 * jnp.sum(s * s - t2, axis=-1, keepdims=True)
  h1 = jnp.maximum(
      jnp.dot(out, w1_ref[...], preferred_element_type=jnp.float32)
      + b1_ref[...], 0.0)
  dnn = jnp.dot(h1, w2_ref[...], preferred_element_type=jnp.float32) + b2_ref[...]
  y = 0.5 * (dnn + fm)
  n = jnp.sqrt(jnp.sum(y * y, axis=-1, keepdims=True))
  y_ref[...] = y / jnp.maximum(n, 1e-12)


def _tower(b, x, statics, de, wl, bl, fmv, fmv2, w1, b1, w2, b2):
  wspec = lambda shp: pl.BlockSpec(shp, lambda i: (0, 0))
  return pl.pallas_call(
      _tower_body,
      grid=(b // _BB,),
      in_specs=[
          pl.BlockSpec((_BB, 64), lambda i: (i, 0)),
      ] + [pl.BlockSpec((_BB, _E), lambda i: (i, 0))] * _NSTATIC + [
          pl.BlockSpec((_BB, _SEQ * _E), lambda i: (i, 0)),
          wspec((64, 32)), wspec((1, 32)),
          wspec((496, _E)), wspec((496, _E)),
          wspec((496, 256)), wspec((1, 256)),
          wspec((256, 64)), wspec((1, 64)),
      ],
      out_specs=pl.BlockSpec((_BB, 64), lambda i: (i, 0)),
      out_shape=jax.ShapeDtypeStruct((b, 64), jnp.float32),
  )(x, *statics, de, wl, bl, fmv, fmv2, w1, b1, w2, b2)


def kernel(user_numerical_features, W_lin, b_lin, city_table, truck_table,
           lcl_table, handling_table, security_table, range_table, scene_table,
           cargo_table, word_table, Wq, Wk, Wv, Wo, ln1_g, ln1_b, ln2_g, ln2_b,
           Wf1, bf1, Wf2, bf2, fm_V, dnn_W1, dnn_b1, dnn_W2, dnn_b2,
           user_search_scene, user_truck_type_labels, user_city_labels,
           user_is_lcl, user_handling_type, user_security_tran,
           user_cargo_category, user_cargo_describe, user_search_range):
  b = user_numerical_features.shape[0]
  i32 = lambda a: a.astype(jnp.int32)

  word_idx = i32(user_cargo_describe).reshape(-1)
  c0 = i32(user_city_labels[:, 0])
  c1 = i32(user_city_labels[:, 1])
  tcols = [i32(user_truck_type_labels[:, j]) for j in range(5)]
  smalls = [i32(user_is_lcl), i32(user_handling_type),
            i32(user_security_tran), i32(user_search_range),
            i32(user_search_scene), i32(user_cargo_category)]

  r2 = lambda a: a.reshape(1, -1)
  wqkv = jnp.concatenate([Wq * 0.25, Wk, Wv], axis=1)
  fm_V2 = fm_V * fm_V

  nh = 2
  hb = b // nh
  ys = []
  for p in range(nh):
    lo = p * hb
    sl = lambda a: lax.slice_in_dim(a, lo, lo + hb, axis=0)
    word_rows, *statics = _sc_gather(
        hb, word_table, city_table, truck_table, lcl_table, handling_table,
        security_table, range_table, scene_table, cargo_table,
        lax.slice_in_dim(word_idx, lo * _SEQ, (lo + hb) * _SEQ, axis=0),
        sl(c0), sl(c1), *[sl(t) for t in tcols], *[sl(s) for s in smalls])
    h2 = _transformer(hb, word_rows, wqkv, Wo, r2(ln1_g), r2(ln1_b),
                      r2(ln2_g), r2(ln2_b), Wf1, r2(bf1), Wf2, r2(bf2))
    desc = h2.reshape(hb, _SEQ * _E)
    ys.append(_tower(hb, sl(user_numerical_features), statics, desc,
                     W_lin, r2(b_lin), fm_V, fm_V2,
                     dnn_W1, r2(dnn_b1), dnn_W2, r2(dnn_b2)))
  return jnp.concatenate(ys, axis=0)


# speedup vs baseline: 1.0094x; 1.0094x over previous
import functools

import jax
import jax.numpy as jnp
from jax import lax
from jax.experimental import pallas as pl
from jax.experimental.pallas import tpu as pltpu
from jax.experimental.pallas import tpu_sc as plsc

_E = 16
_SEQ = 20
_NC, _NS = 2, 16
_NW = _NC * _NS
_CHUNK = 128
_WBUF = 2560
_BA = 512
_G = 8
_BB = 512
_NSTATIC = 13


def _sc_gather_body(word_table, city_table, truck_table, lcl_table,
                    handling_table, security_table, range_table, scene_table,
                    cargo_table, word_idx, c0, c1, t0, t1, t2, t3, t4,
                    s_lcl, s_han, s_sec, s_rng, s_scn, s_cgo,
                    word_out, *rest):
  static_outs = rest[:_NSTATIC]
  widx_v, wrows_v, sidx_v, srows_v, sem = rest[_NSTATIC:]
  b = static_outs[0].shape[0]
  r = b // _NW
  rw = (b * _SEQ) // _NW
  wid = lax.axis_index("s") * _NC + lax.axis_index("c")
  base = wid * r
  wbase = wid * rw

  tasks = [(city_table, c0), (city_table, c1),
           (truck_table, t0), (truck_table, t1), (truck_table, t2),
           (truck_table, t3), (truck_table, t4),
           (lcl_table, s_lcl), (handling_table, s_han),
           (security_table, s_sec), (range_table, s_rng),
           (scene_table, s_scn), (cargo_table, s_cgo)]
  for (tab, idx_hbm), st_out in zip(tasks, static_outs):
    pltpu.sync_copy(idx_hbm.at[pl.ds(base, r)], sidx_v)
    cps = []
    for j in range(r // _CHUNK):
      cps.append(pltpu.async_copy(
          tab.at[sidx_v.at[pl.ds(j * _CHUNK, _CHUNK)]],
          srows_v.at[pl.ds(j * _CHUNK, _CHUNK)], sem))
    for c in cps:
      c.wait()
    pltpu.sync_copy(srows_v, st_out.at[pl.ds(base, r)])

  pltpu.sync_copy(word_idx.at[pl.ds(wbase, rw)], widx_v)

  def body(g, carry):
    off = g * _WBUF
    cps = []
    for j in range(_WBUF // _CHUNK):
      cps.append(pltpu.async_copy(
          word_table.at[widx_v.at[pl.ds(off + j * _CHUNK, _CHUNK)]],
          wrows_v.at[pl.ds(j * _CHUNK, _CHUNK)], sem))
    for c in cps:
      c.wait()
    pltpu.sync_copy(wrows_v, word_out.at[pl.ds(wbase + off, _WBUF)])
    return carry

  lax.fori_loop(0, rw // _WBUF, body, 0)


def _sc_gather(b, word_table, city_table, truck_table, lcl_table,
               handling_table, security_table, range_table, scene_table,
               cargo_table, word_idx, c0, c1, t0, t1, t2, t3, t4,
               s_lcl, s_han, s_sec, s_rng, s_scn, s_cgo):
  r = b // _NW
  rw = (b * _SEQ) // _NW
  fn = pl.kernel(
      _sc_gather_body,
      out_type=[jax.ShapeDtypeStruct((b * _SEQ, _E), jnp.float32)] +
               [jax.ShapeDtypeStruct((b, _E), jnp.float32)] * _NSTATIC,
      mesh=plsc.VectorSubcoreMesh(core_axis_name="c", subcore_axis_name="s"),
      scratch_types=[
          pltpu.VMEM((rw,), jnp.int32),
          pltpu.VMEM((_WBUF, _E), jnp.float32),
          pltpu.VMEM((r,), jnp.int32),
          pltpu.VMEM((r, _E), jnp.float32),
          pltpu.SemaphoreType.DMA,
      ],
      compiler_params=pltpu.CompilerParams(use_tc_tiling_on_sc=False),
  )
  return fn(word_table, city_table, truck_table, lcl_table, handling_table,
            security_table, range_table, scene_table, cargo_table,
            word_idx, c0, c1, t0, t1, t2, t3, t4,
            s_lcl, s_han, s_sec, s_rng, s_scn, s_cgo)


def _layer_norm(x, g, bta):
  m = jnp.mean(x, axis=-1, keepdims=True)
  d = x - m
  v = jnp.mean(d * d, axis=-1, keepdims=True)
  return d / jnp.sqrt(v + 1e-5) * g + bta


def _tf_body(h_ref, wqkv_ref, wo_ref, ln1g_ref, ln1b_ref,
             ln2g_ref, ln2b_ref, wf1_ref, bf1_ref, wf2_ref, bf2_ref,
             out_ref, q_s, k_s, v_s, e_s):
  h = h_ref[...]
  qkv = jnp.dot(h, wqkv_ref[...], preferred_element_type=jnp.float32)
  q_s[...] = qkv[:, 0:_E]
  k_s[...] = qkv[:, _E:2 * _E]
  v_s[...] = qkv[:, 2 * _E:3 * _E]

  gl = _G * _SEQ
  n = _BA * _SEQ
  ngrp = _BA // _G

  def sbody(g, carry):
    off = g * gl
    e_s[pl.ds(off, gl), :] = lax.dot_general(
        q_s[pl.ds(off, gl), :], k_s[pl.ds(off, gl), :],
        (((1,), (1,)), ((), ())), preferred_element_type=jnp.float32)
    return carry

  lax.fori_loop(0, ngrp, sbody, 0, unroll=4)

  ri = (lax.broadcasted_iota(jnp.int32, (n, gl), 0) // _SEQ) % _G
  ci = lax.broadcasted_iota(jnp.int32, (n, gl), 1) // _SEQ
  e = jnp.where(ri == ci, jnp.exp(e_s[...]), 0.0)
  r = 1.0 / jnp.sum(e, axis=-1, keepdims=True)
  e_s[...] = e

  def obody(g, carry):
    off = g * gl
    q_s[pl.ds(off, gl), :] = jnp.dot(
        e_s[pl.ds(off, gl), :], v_s[pl.ds(off, gl), :],
        preferred_element_type=jnp.float32)
    return carry

  lax.fori_loop(0, ngrp, obody, 0, unroll=4)

  o = jnp.dot(q_s[...] * r, wo_ref[...], preferred_element_type=jnp.float32)
  h1 = _layer_norm(h + o, ln1g_ref[...], ln1b_ref[...])
  f = jnp.maximum(
      jnp.dot(h1, wf1_ref[...], preferred_element_type=jnp.float32)
      + bf1_ref[...], 0.0)
  f = jnp.dot(f, wf2_ref[...], preferred_element_type=jnp.float32) + bf2_ref[...]
  out_ref[...] = _layer_norm(h1 + f, ln2g_ref[...], ln2b_ref[...])


def _transformer(b, h, wqkv, wo, ln1g, ln1b, ln2g, ln2b,
                 wf1, bf1, wf2, bf2):
  n = _BA * _SEQ
  wspec = lambda shp: pl.BlockSpec(shp, lambda i: (0, 0))
  return pl.pallas_call(
      _tf_body,
      grid=(b // _BA,),
      in_specs=[
          pl.BlockSpec((n, _E), lambda i: (i, 0)),
          wspec((_E, 3 * _E)), wspec((_E, _E)),
          wspec((1, _E)), wspec((1, _E)), wspec((1, _E)), wspec((1, _E)),
          wspec((_E, 64)), wspec((1, 64)), wspec((64, _E)), wspec((1, _E)),
      ],
      out_specs=pl.BlockSpec((n, _E), lambda i: (i, 0)),
      out_shape=jax.ShapeDtypeStruct((b * _SEQ, _E), jnp.float32),
      scratch_shapes=[pltpu.VMEM((n, _E), jnp.float32),
                      pltpu.VMEM((n, _E), jnp.float32),
                      pltpu.VMEM((n, _E), jnp.float32),
                      pltpu.VMEM((n, _G * _SEQ), jnp.float32)],
  )(h, wqkv, wo, ln1g, ln1b, ln2g, ln2b, wf1, bf1, wf2, bf2)


def _tower_body(x_ref, c0_ref, c1_ref, t0_ref, t1_ref, t2_ref, t3_ref, t4_ref,
                lcl_ref, han_ref, sec_ref, rng_ref, scn_ref, cgo_ref,
                de_ref, wl_ref, bl_ref, fmv_ref, fmv2_ref,
                w1_ref, b1_ref, w2_ref, b2_ref, y_ref):
  scale = 0.125
  num = jnp.dot(x_ref[...], wl_ref[...] * scale,
                preferred_element_type=jnp.float32) + bl_ref[...]
  tm = (t0_ref[...] + t1_ref[...] + t2_ref[...] + t3_ref[...]
        + t4_ref[...]) * 0.2
  de = de_ref[...]
  out = jnp.concatenate(
      [num, c0_ref[...], c1_ref[...], tm, lcl_ref[...], han_ref[...],
       sec_ref[...], rng_ref[...], scn_ref[...], cgo_ref[...], de],
      axis=1)
  s = jnp.dot(out, fmv_ref[...], preferred_element_type=jnp.float32)
  t2 = jnp.dot(out * out, fmv2_ref[...], preferred_element_type=jnp.float32)
  fm = 0.5 * jnp.sum(s * s - t2, axis=-1, keepdims=True)
  h1 = jnp.maximum(
      jnp.dot(out, w1_ref[...], preferred_element_type=jnp.float32)
      + b1_ref[...], 0.0)
  dnn = jnp.dot(h1, w2_ref[...], preferred_element_type=jnp.float32) + b2_ref[...]
  y = 0.5 * (dnn + fm)
  n = jnp.sqrt(jnp.sum(y * y, axis=-1, keepdims=True))
  y_ref[...] = y / jnp.maximum(n, 1e-12)


def _tower(b, x, statics, de, wl, bl, fmv, fmv2, w1, b1, w2, b2):
  wspec = lambda shp: pl.BlockSpec(shp, lambda i: (0, 0))
  return pl.pallas_call(
      _tower_body,
      grid=(b // _BB,),
      in_specs=[
          pl.BlockSpec((_BB, 64), lambda i: (i, 0)),
      ] + [pl.BlockSpec((_BB, _E), lambda i: (i, 0))] * _NSTATIC + [
          pl.BlockSpec((_BB, _SEQ * _E), lambda i: (i, 0)),
          wspec((64, 32)), wspec((1, 32)),
          wspec((496, _E)), wspec((496, _E)),
          wspec((496, 256)), wspec((1, 256)),
          wspec((256, 64)), wspec((1, 64)),
      ],
      out_specs=pl.BlockSpec((_BB, 64), lambda i: (i, 0)),
      out_shape=jax.ShapeDtypeStruct((b, 64), jnp.float32),
  )(x, *statics, de, wl, bl, fmv, fmv2, w1, b1, w2, b2)


def kernel(user_numerical_features, W_lin, b_lin, city_table, truck_table,
           lcl_table, handling_table, security_table, range_table, scene_table,
           cargo_table, word_table, Wq, Wk, Wv, Wo, ln1_g, ln1_b, ln2_g, ln2_b,
           Wf1, bf1, Wf2, bf2, fm_V, dnn_W1, dnn_b1, dnn_W2, dnn_b2,
           user_search_scene, user_truck_type_labels, user_city_labels,
           user_is_lcl, user_handling_type, user_security_tran,
           user_cargo_category, user_cargo_describe, user_search_range):
  b = user_numerical_features.shape[0]
  i32 = lambda a: a.astype(jnp.int32)

  word_idx = i32(user_cargo_describe).reshape(-1)
  c0 = i32(user_city_labels[:, 0])
  c1 = i32(user_city_labels[:, 1])
  tcols = [i32(user_truck_type_labels[:, j]) for j in range(5)]
  smalls = [i32(user_is_lcl), i32(user_handling_type),
            i32(user_security_tran), i32(user_search_range),
            i32(user_search_scene), i32(user_cargo_category)]

  r2 = lambda a: a.reshape(1, -1)
  wqkv = jnp.concatenate([Wq * 0.25, Wk, Wv], axis=1)
  fm_V2 = fm_V * fm_V

  nh = 4
  hb = b // nh
  ys = []
  for p in range(nh):
    lo = p * hb
    sl = lambda a: lax.slice_in_dim(a, lo, lo + hb, axis=0)
    word_rows, *statics = _sc_gather(
        hb, word_table, city_table, truck_table, lcl_table, handling_table,
        security_table, range_table, scene_table, cargo_table,
        lax.slice_in_dim(word_idx, lo * _SEQ, (lo + hb) * _SEQ, axis=0),
        sl(c0), sl(c1), *[sl(t) for t in tcols], *[sl(s) for s in smalls])
    h2 = _transformer(hb, word_rows, wqkv, Wo, r2(ln1_g), r2(ln1_b),
                      r2(ln2_g), r2(ln2_b), Wf1, r2(bf1), Wf2, r2(bf2))
    desc = h2.reshape(hb, _SEQ * _E)
    ys.append(_tower(hb, sl(user_numerical_features), statics, desc,
                     W_lin, r2(b_lin), fm_V, fm_V2,
                     dnn_W1, r2(dnn_b1), dnn_W2, r2(dnn_b2)))
  return jnp.concatenate(ys, axis=0)
